# baseline (speedup 1.0000x reference)
import functools

import jax
import jax.numpy as jnp
from jax.experimental import pallas as pl
from jax.experimental.pallas import tpu as pltpu

_R = 512
_NS = 4


def _stream_mlp(c, xcols_get, w1, w1p, w2, m, *, rows, S, D):
    nchunks = S // 2
    acc = jnp.zeros((rows, 2 * D), jnp.float32)
    for j in range(nchunks):
        xj = xcols_get(j)
        sn = jnp.tanh(jax.lax.dot(xj, w1p,
                                  preferred_element_type=jnp.float32))
        acc = acc + sn * xj
    pn = acc[:, :D] + acc[:, D:]
    sc = jnp.tanh(jax.lax.dot(c, w1, preferred_element_type=jnp.float32))
    t = pn + sc * c
    agg = jax.lax.dot(t, w2, preferred_element_type=jnp.float32)
    return jnp.where(m > 0.0, c, agg)


def _mlp_body(*refs, rows, S, D, ns):
    c_refs = refs[:ns]
    n_refs = refs[ns:2 * ns]
    w1 = refs[2 * ns][...]
    w1p = refs[2 * ns + 1][...]
    w2 = refs[2 * ns + 2][...]
    m_refs = refs[2 * ns + 3:3 * ns + 3]
    o_refs = refs[3 * ns + 3:]
    for k in range(ns):
        n_ref = n_refs[k]
        get = lambda j, n_ref=n_ref: n_ref[:, 2 * D * j:2 * D * (j + 1)]
        o_refs[k][...] = _stream_mlp(
            c_refs[k][...], get, w1, w1p, w2, m_refs[k][...],
            rows=rows, S=S, D=D)


def _mlp_all_rows(center_feat, neighbor_rows, W1, W1p, W2, is_neg, *,
                  interpret=False):
    B, D = center_feat.shape
    SD = neighbor_rows.shape[1]
    S = SD // D
    R, NS = _R, _NS
    steps = B // (R * NS)
    body = functools.partial(_mlp_body, rows=R, S=S, D=D, ns=NS)

    def rowmap(k, bs):
        return lambda i, k=k, bs=bs: (i + k * bs, 0)

    in_specs = (
        [pl.BlockSpec((R, D), rowmap(k, steps)) for k in range(NS)]
        + [pl.BlockSpec((R, SD), rowmap(k, steps)) for k in range(NS)]
        + [pl.BlockSpec((D, D), lambda i: (0, 0)),
           pl.BlockSpec((2 * D, 2 * D), lambda i: (0, 0)),
           pl.BlockSpec((D, D), lambda i: (0, 0))]
        + [pl.BlockSpec((R, 1), rowmap(k, steps)) for k in range(NS)]
    )
    out_specs = [pl.BlockSpec((R, D), rowmap(k, steps)) for k in range(NS)]
    outs = pl.pallas_call(
        body,
        grid=(steps,),
        in_specs=in_specs,
        out_specs=out_specs,
        out_shape=[jax.ShapeDtypeStruct((B, D), jnp.float32)
                   for _ in range(NS)],
        compiler_params=pltpu.CompilerParams(
            dimension_semantics=("arbitrary",),
        ),
        interpret=interpret,
    )(*([center_feat] * NS), *([neighbor_rows] * NS), W1, W1p, W2,
      *([is_neg] * NS))
    Bq = B // NS
    return jnp.concatenate([o[k * Bq:(k + 1) * Bq] for k, o in enumerate(outs)],
                           axis=0)


def kernel(center_feat, neighbor_feats, W1, W2):
    B, D = center_feat.shape
    S = neighbor_feats.shape[1]
    ano = int(B * 0.1)
    batch_center = jnp.mean(center_feat, axis=-1)
    diff_center = jnp.sum(center_feat - batch_center[:, None], axis=-1)
    sorted_idx = jnp.argsort(diff_center)
    neg_idx = sorted_idx[B - ano:]
    is_neg = jnp.zeros((B,), jnp.float32).at[neg_idx].set(1.0)[:, None]
    W1p = jnp.zeros((2 * D, 2 * D), jnp.float32)
    W1p = W1p.at[:D, :D].set(W1).at[D:, D:].set(W1)
    out = _mlp_all_rows(center_feat, neighbor_feats.reshape(B, S * D),
                        W1, W1p, W2, is_neg)
    return out, neg_idx


# speedup vs baseline: 1.0528x; 1.0528x over previous
import functools

import jax
import jax.numpy as jnp
from jax.experimental import pallas as pl
from jax.experimental.pallas import tpu as pltpu

_R = 512


def _mlp_body(c_ref, n_ref, w1_ref, w1p_ref, w2_ref, m_ref, o_ref, *,
              rows, S, D):
    c = c_ref[...]
    w1p = w1p_ref[...]
    nchunks = S * D // (2 * D)
    acc = jnp.zeros((rows, 2 * D), jnp.float32)
    for j in range(nchunks):
        xj = n_ref[:, 2 * D * j:2 * D * (j + 1)]
        sn = jnp.tanh(jax.lax.dot(xj, w1p,
                                  preferred_element_type=jnp.float32))
        acc = acc + sn * xj
    pn = acc[:, :D] + acc[:, D:]
    w1 = w1_ref[...]
    sc = jnp.tanh(jax.lax.dot(c, w1, preferred_element_type=jnp.float32))
    t = pn + sc * c
    agg = jax.lax.dot(t, w2_ref[...], preferred_element_type=jnp.float32)
    m = m_ref[...]
    o_ref[...] = jnp.where(m > 0.0, c, agg)


def _mlp_all_rows(center_feat, neighbor_rows, W1, W1p, W2, is_neg, *,
                  interpret=False):
    B, D = center_feat.shape
    SD = neighbor_rows.shape[1]
    S = SD // D
    R = _R
    body = functools.partial(_mlp_body, rows=R, S=S, D=D)
    return pl.pallas_call(
        body,
        grid=(B // R,),
        in_specs=[
            pl.BlockSpec((R, D), lambda i: (i, 0)),
            pl.BlockSpec((R, SD), lambda i: (i, 0)),
            pl.BlockSpec((D, D), lambda i: (0, 0)),
            pl.BlockSpec((2 * D, 2 * D), lambda i: (0, 0)),
            pl.BlockSpec((D, D), lambda i: (0, 0)),
            pl.BlockSpec((R, 1), lambda i: (i, 0)),
        ],
        out_specs=pl.BlockSpec((R, D), lambda i: (i, 0)),
        out_shape=jax.ShapeDtypeStruct((B, D), jnp.float32),
        compiler_params=pltpu.CompilerParams(
            dimension_semantics=("parallel",),
        ),
        interpret=interpret,
    )(center_feat, neighbor_rows, W1, W1p, W2, is_neg)


def kernel(center_feat, neighbor_feats, W1, W2):
    B, D = center_feat.shape
    S = neighbor_feats.shape[1]
    ano = int(B * 0.1)
    batch_center = jnp.mean(center_feat, axis=-1)
    diff_center = jnp.sum(center_feat - batch_center[:, None], axis=-1)
    sorted_idx = jnp.argsort(diff_center)
    neg_idx = sorted_idx[B - ano:]
    is_neg = jnp.zeros((B,), jnp.float32).at[neg_idx].set(1.0)[:, None]
    W1p = jnp.zeros((2 * D, 2 * D), jnp.float32)
    W1p = W1p.at[:D, :D].set(W1).at[D:, D:].set(W1)
    out = _mlp_all_rows(center_feat, neighbor_feats.reshape(B, S * D),
                        W1, W1p, W2, is_neg)
    return out, neg_idx
